# Initial kernel scaffold; baseline (speedup 1.0000x reference)
import jax
import jax.numpy as jnp
from jax.experimental import pallas as pl


def kernel(i, j, relative_attention_bias):
    raise NotImplementedError("write your pallas kernel here")


# speedup vs baseline: 55.5532x; 55.5532x over previous
import jax
import jax.numpy as jnp
from jax import lax
from jax.experimental import pallas as pl
from jax.experimental.pallas import tpu as pltpu
from jax.experimental.pallas import tpu_sc as plsc

HEADS = 12
SEQ = 2048
NUM_BUCKETS = 32
ULEN = 4096
ROWS_PER_UNIT = SEQ // 8
THRESHOLDS = (19, 21, 24, 27, 31, 35, 40, 46, 52, 59, 67, 77, 87, 99, 113)


def _bias_body(table_hbm, out_hbm, table_v, u0, u1, u2, sem):
    cid = lax.axis_index("c")
    sid = lax.axis_index("s")
    wid = sid * 2 + cid

    pltpu.sync_copy(table_hbm, table_v)

    lane = lax.iota(jnp.int32, 16)
    one = jnp.zeros((16,), jnp.int32) + 1
    zero = jnp.zeros((16,), jnp.int32)
    ubufs = (u0, u1, u2)

    for m in range(3):
        uid = wid + 32 * m
        h = uid // 8
        r = uid % 8
        sft = 7 - r
        u = ubufs[m]
        base_n = 2047 - sft
        hv = zero + h

        def build(c, carry, u=u, base_n=base_n, hv=hv):
            t0 = c * 16
            n = jnp.maximum(base_n - (t0 + lane), 0)
            b = zero + 16
            for th in THRESHOLDS:
                b = b + jnp.where(n >= th, one, zero)
            b = jnp.where(n < 16, n, b)
            u[pl.ds(t0, 16)] = plsc.load_gather(table_v, [b * HEADS + hv])
            return carry

        lax.fori_loop(0, ULEN // 16, build, 0)

        def fire(k, carry, u=u, h=h, r=r):
            o = pl.multiple_of(2040 - 8 * k, 8)
            dst = pl.multiple_of((h * SEQ + r + 8 * k) * SEQ, 8)
            pltpu.async_copy(u.at[pl.ds(o, SEQ)], out_hbm.at[pl.ds(dst, SEQ)], sem)
            return carry

        lax.fori_loop(0, ROWS_PER_UNIT, fire, 0)

    def drain(k, carry):
        pltpu.make_async_copy(
            u0.at[pl.ds(0, SEQ)], out_hbm.at[pl.ds(0, SEQ)], sem
        ).wait()
        return carry

    lax.fori_loop(0, 3 * ROWS_PER_UNIT, drain, 0)


def kernel(i, j, relative_attention_bias):
    del i, j
    mesh = plsc.VectorSubcoreMesh(core_axis_name="c", subcore_axis_name="s")
    run = pl.kernel(
        _bias_body,
        out_type=jax.ShapeDtypeStruct((HEADS * SEQ * SEQ,), jnp.float32),
        mesh=mesh,
        compiler_params=pltpu.CompilerParams(needs_layout_passes=False),
        scratch_types=[
            pltpu.VMEM((NUM_BUCKETS * HEADS,), jnp.float32),
            pltpu.VMEM((ULEN,), jnp.float32),
            pltpu.VMEM((ULEN,), jnp.float32),
            pltpu.VMEM((ULEN,), jnp.float32),
            pltpu.SemaphoreType.DMA,
        ],
    )
    flat = run(relative_attention_bias.reshape(-1))
    return flat.reshape(HEADS, SEQ, SEQ)


# speedup vs baseline: 55.7011x; 1.0027x over previous
import jax
import jax.numpy as jnp
from jax import lax
from jax.experimental import pallas as pl
from jax.experimental.pallas import tpu as pltpu
from jax.experimental.pallas import tpu_sc as plsc

HEADS = 12
SEQ = 2048
NUM_BUCKETS = 32
ULEN = 4096
ROWS_PER_UNIT = SEQ // 16
UNITS = 6
THRESHOLDS = (19, 21, 24, 27, 31, 35, 40, 46, 52, 59, 67, 77, 87, 99, 113)


def _bias_body(table_hbm, out_hbm, table_v, u0, u1, u2, u3, u4, u5, sem):
    cid = lax.axis_index("c")
    sid = lax.axis_index("s")
    wid = sid * 2 + cid
    r = wid % 16
    h0 = wid // 16
    base_n = 2032 + r

    pltpu.sync_copy(table_hbm, table_v)

    lane = lax.iota(jnp.int32, 16)
    one = jnp.zeros((16,), jnp.int32) + 1
    zero = jnp.zeros((16,), jnp.int32)

    ubufs = (u0, u1, u2, u3, u4, u5)
    for m in range(UNITS):
        h = h0 + 2 * m
        hv = zero + h
        u = ubufs[m]

        c31 = plsc.load_gather(table_v, [hv + (NUM_BUCKETS - 1) * HEADS])
        c0 = plsc.load_gather(table_v, [hv])

        def splat31(c, carry, u=u, c31=c31):
            u[pl.ds(c * 16, 16)] = c31
            return carry

        lax.fori_loop(0, 120, splat31, 0)

        def splat0(c, carry, u=u, c0=c0):
            u[pl.ds(c * 16, 16)] = c0
            return carry

        lax.fori_loop(128, ULEN // 16, splat0, 0)

        def build(c, carry, u=u, hv=hv):
            t0 = c * 16
            n = jnp.maximum(base_n - (t0 + lane), 0)
            b = zero + 16
            for th in THRESHOLDS:
                b = b + jnp.where(n >= th, one, zero)
            b = jnp.where(n < 16, n, b)
            u[pl.ds(t0, 16)] = plsc.load_gather(table_v, [b * HEADS + hv])
            return carry

        lax.fori_loop(120, 128, build, 0)

        def fire(k, carry, u=u, h=h):
            o = pl.multiple_of(2032 - 16 * k, 16)
            dst = pl.multiple_of((h * SEQ + r + 16 * k) * SEQ, 8)
            pltpu.async_copy(u.at[pl.ds(o, SEQ)], out_hbm.at[pl.ds(dst, SEQ)], sem)
            return carry

        lax.fori_loop(0, ROWS_PER_UNIT, fire, 0)

    def drain(k, carry):
        pltpu.make_async_copy(
            u0.at[pl.ds(0, SEQ)], out_hbm.at[pl.ds(0, SEQ)], sem
        ).wait()
        return carry

    lax.fori_loop(0, UNITS * ROWS_PER_UNIT, drain, 0)


def kernel(i, j, relative_attention_bias):
    del i, j
    mesh = plsc.VectorSubcoreMesh(core_axis_name="c", subcore_axis_name="s")
    run = pl.kernel(
        _bias_body,
        out_type=jax.ShapeDtypeStruct((HEADS * SEQ * SEQ,), jnp.float32),
        mesh=mesh,
        compiler_params=pltpu.CompilerParams(needs_layout_passes=False),
        scratch_types=[
            pltpu.VMEM((NUM_BUCKETS * HEADS,), jnp.float32),
            pltpu.VMEM((ULEN,), jnp.float32),
            pltpu.VMEM((ULEN,), jnp.float32),
            pltpu.VMEM((ULEN,), jnp.float32),
            pltpu.VMEM((ULEN,), jnp.float32),
            pltpu.VMEM((ULEN,), jnp.float32),
            pltpu.VMEM((ULEN,), jnp.float32),
            pltpu.SemaphoreType.DMA,
        ],
    )
    flat = run(relative_attention_bias.reshape(-1))
    return flat.reshape(HEADS, SEQ, SEQ)


# speedup vs baseline: 160.5991x; 2.8832x over previous
import jax
import jax.numpy as jnp
from jax import lax
from jax.experimental import pallas as pl
from jax.experimental.pallas import tpu as pltpu
from jax.experimental.pallas import tpu_sc as plsc

HEADS = 12
SEQ = 2048
NUM_BUCKETS = 32
VLEN = 4096
NBUF = 3
UNITS = 6
THRESHOLDS = (19, 21, 24, 27, 31, 35, 40, 46, 52, 59, 67, 77, 87, 99, 113)


def _bias_body(table_hbm, out_hbm, table_v, v0, v1, v2, sem):
    cid = lax.axis_index("c")
    sid = lax.axis_index("s")
    wid = sid * 2 + cid
    q = wid % 16
    h0 = wid // 16

    pltpu.sync_copy(table_hbm, table_v)

    lane = lax.iota(jnp.int32, 16)
    one = jnp.zeros((16,), jnp.int32) + 1
    zero = jnp.zeros((16,), jnp.int32)
    vbufs = (v0, v1, v2)

    def fire_unit(V, h):
        for s in range(16):
            row0 = pl.multiple_of(128 * s + 8 * q, 8)
            pltpu.async_copy(
                V.at[:, pl.ds(1920 - 128 * s, SEQ)],
                out_hbm.at[h, pl.ds(row0, 8), :],
                sem,
            )

    def drain_unit():
        for s in range(16):
            pltpu.make_async_copy(
                v0.at[:, pl.ds(0, SEQ)],
                out_hbm.at[0, pl.ds(0, 8), :],
                sem,
            ).wait()

    def build_unit(V, h):
        hv = zero + h
        c31 = plsc.load_gather(table_v, [hv + (NUM_BUCKETS - 1) * HEADS])
        c0 = plsc.load_gather(table_v, [hv])
        for rr in range(8):
            n0 = 1920 + 8 * q + rr

            def splat31(c, carry, V=V, rr=rr, c31=c31):
                V[rr, pl.ds(c * 16, 16)] = c31
                return carry

            lax.fori_loop(0, 112, splat31, 0)

            def splat0(c, carry, V=V, rr=rr, c0=c0):
                V[rr, pl.ds(c * 16, 16)] = c0
                return carry

            lax.fori_loop(128, VLEN // 16, splat0, 0)

            def build(c, carry, V=V, rr=rr, n0=n0, hv=hv):
                t0 = c * 16
                n = jnp.maximum(n0 - (t0 + lane), 0)
                b = zero + 16
                for th in THRESHOLDS:
                    b = b + jnp.where(n >= th, one, zero)
                b = jnp.where(n < 16, n, b)
                V[rr, pl.ds(t0, 16)] = plsc.load_gather(
                    table_v, [b * HEADS + hv]
                )
                return carry

            lax.fori_loop(112, 128, build, 0)

    for m in range(UNITS):
        if m >= NBUF:
            drain_unit()
        h = h0 + 2 * m
        build_unit(vbufs[m % NBUF], h)
        fire_unit(vbufs[m % NBUF], h)
    for m in range(NBUF):
        drain_unit()


def kernel(i, j, relative_attention_bias):
    del i, j
    mesh = plsc.VectorSubcoreMesh(core_axis_name="c", subcore_axis_name="s")
    run = pl.kernel(
        _bias_body,
        out_type=jax.ShapeDtypeStruct((HEADS, SEQ, SEQ), jnp.float32),
        mesh=mesh,
        compiler_params=pltpu.CompilerParams(needs_layout_passes=False),
        scratch_types=[
            pltpu.VMEM((NUM_BUCKETS * HEADS,), jnp.float32),
            pltpu.VMEM((8, VLEN), jnp.float32),
            pltpu.VMEM((8, VLEN), jnp.float32),
            pltpu.VMEM((8, VLEN), jnp.float32),
            pltpu.SemaphoreType.DMA,
        ],
    )
    return run(relative_attention_bias.reshape(-1))


# speedup vs baseline: 167.7797x; 1.0447x over previous
import jax
import jax.numpy as jnp
from jax import lax
from jax.experimental import pallas as pl
from jax.experimental.pallas import tpu as pltpu
from jax.experimental.pallas import tpu_sc as plsc

HEADS = 12
SEQ = 2048
NUM_BUCKETS = 32
VLEN = 4096
NBUF = 3
UNITS = 6
THRESHOLDS = (19, 21, 24, 27, 31, 35, 40, 46, 52, 59, 67, 77, 87, 99, 113)


def _bias_body(table_hbm, out_hbm, table_v, v0, v1, v2, sem):
    cid = lax.axis_index("c")
    sid = lax.axis_index("s")
    wid = sid * 2 + cid
    q = wid % 16
    h0 = wid // 16

    pltpu.sync_copy(table_hbm, table_v)

    lane = lax.iota(jnp.int32, 16)
    one = jnp.zeros((16,), jnp.int32) + 1
    zero = jnp.zeros((16,), jnp.int32)
    vbufs = (v0, v1, v2)

    def fire_unit(V, h):
        def fire(s, carry, V=V, h=h):
            col = pl.multiple_of(1920 - 128 * s, 128)
            row0 = pl.multiple_of(128 * s + 8 * q, 8)
            pltpu.async_copy(
                V.at[:, pl.ds(col, SEQ)],
                out_hbm.at[h, pl.ds(row0, 8), :],
                sem,
            )
            return carry

        lax.fori_loop(0, 16, fire, 0)

    def drain_unit():
        def drain(s, carry):
            pltpu.make_async_copy(
                v0.at[:, pl.ds(0, SEQ)],
                out_hbm.at[0, pl.ds(0, 8), :],
                sem,
            ).wait()
            return carry

        lax.fori_loop(0, 16, drain, 0)

    def build_unit(V, h):
        hv = zero + h
        c31 = plsc.load_gather(table_v, [hv + (NUM_BUCKETS - 1) * HEADS])
        c0 = plsc.load_gather(table_v, [hv])
        for rr in range(8):
            n0 = 1920 + 8 * q + rr

            def splat31(c, carry, V=V, rr=rr, c31=c31):
                for jj in range(4):
                    V[rr, pl.ds((c * 4 + jj) * 16, 16)] = c31
                return carry

            lax.fori_loop(0, 28, splat31, 0)

            def splat0(c, carry, V=V, rr=rr, c0=c0):
                for jj in range(4):
                    V[rr, pl.ds((c * 4 + jj) * 16, 16)] = c0
                return carry

            lax.fori_loop(32, VLEN // 64, splat0, 0)

            def build(c, carry, V=V, rr=rr, n0=n0, hv=hv):
                t0 = c * 16
                n = jnp.maximum(n0 - (t0 + lane), 0)
                b = zero + 16
                for th in THRESHOLDS:
                    b = b + jnp.where(n >= th, one, zero)
                b = jnp.where(n < 16, n, b)
                V[rr, pl.ds(t0, 16)] = plsc.load_gather(
                    table_v, [b * HEADS + hv]
                )
                return carry

            lax.fori_loop(112, 128, build, 0)

    for m in range(UNITS):
        if m >= NBUF:
            drain_unit()
        h = h0 + 2 * m
        build_unit(vbufs[m % NBUF], h)
        fire_unit(vbufs[m % NBUF], h)
    for m in range(NBUF):
        drain_unit()


def kernel(i, j, relative_attention_bias):
    del i, j
    mesh = plsc.VectorSubcoreMesh(core_axis_name="c", subcore_axis_name="s")
    run = pl.kernel(
        _bias_body,
        out_type=jax.ShapeDtypeStruct((HEADS, SEQ, SEQ), jnp.float32),
        mesh=mesh,
        compiler_params=pltpu.CompilerParams(needs_layout_passes=False),
        scratch_types=[
            pltpu.VMEM((NUM_BUCKETS * HEADS,), jnp.float32),
            pltpu.VMEM((8, VLEN), jnp.float32),
            pltpu.VMEM((8, VLEN), jnp.float32),
            pltpu.VMEM((8, VLEN), jnp.float32),
            pltpu.SemaphoreType.DMA,
        ],
    )
    return run(relative_attention_bias.reshape(-1))


# speedup vs baseline: 184.1087x; 1.0973x over previous
import jax
import jax.numpy as jnp
from jax import lax
from jax.experimental import pallas as pl
from jax.experimental.pallas import tpu as pltpu
from jax.experimental.pallas import tpu_sc as plsc

HEADS = 12
SEQ = 2048
NUM_BUCKETS = 32
VLEN = 4096
NBUF = 3
UNITS = 6
THRESHOLDS = (19, 21, 24, 27, 31, 35, 40, 46, 52, 59, 67, 77, 87, 99, 113)


def _bias_body(table_hbm, out_hbm, table_v, v0, v1, v2, sem):
    cid = lax.axis_index("c")
    sid = lax.axis_index("s")
    wid = sid * 2 + cid
    q = wid % 16
    h0 = wid // 16

    pltpu.sync_copy(table_hbm, table_v)

    lane = lax.iota(jnp.int32, 16)
    one = jnp.zeros((16,), jnp.int32) + 1
    zero = jnp.zeros((16,), jnp.int32)
    vbufs = (v0, v1, v2)

    def fire_unit(V, h):
        def fire(s, carry, V=V, h=h):
            col = pl.multiple_of(1920 - 128 * s, 128)
            row0 = pl.multiple_of(128 * s + 8 * q, 8)
            pltpu.async_copy(
                V.at[:, pl.ds(col, SEQ)],
                out_hbm.at[h, pl.ds(row0, 8), :],
                sem,
            )
            return carry

        lax.fori_loop(0, 16, fire, 0)

    def drain_unit():
        def drain(s, carry):
            pltpu.make_async_copy(
                v0.at[:, pl.ds(0, SEQ)],
                out_hbm.at[0, pl.ds(0, 8), :],
                sem,
            ).wait()
            return carry

        lax.fori_loop(0, 16, drain, 0)

    def build_unit(V, h):
        hv = zero + h
        c31 = plsc.load_gather(table_v, [hv + (NUM_BUCKETS - 1) * HEADS])
        c0 = plsc.load_gather(table_v, [hv])
        def row(rr, carry, V=V, hv=hv, c31=c31, c0=c0):
            n0 = 1920 + 8 * q + rr

            def splat31(c, carry2, V=V, rr=rr, c31=c31):
                for jj in range(4):
                    V[rr, pl.ds((c * 4 + jj) * 16, 16)] = c31
                return carry2

            lax.fori_loop(0, 28, splat31, 0)

            def splat0(c, carry2, V=V, rr=rr, c0=c0):
                for jj in range(4):
                    V[rr, pl.ds((c * 4 + jj) * 16, 16)] = c0
                return carry2

            lax.fori_loop(32, VLEN // 64, splat0, 0)

            def build(c, carry2, V=V, rr=rr, n0=n0, hv=hv):
                t0 = c * 16
                n = jnp.maximum(n0 - (t0 + lane), 0)
                b = zero + 16
                for th in THRESHOLDS:
                    b = b + jnp.where(n >= th, one, zero)
                b = jnp.where(n < 16, n, b)
                V[rr, pl.ds(t0, 16)] = plsc.load_gather(
                    table_v, [b * HEADS + hv]
                )
                return carry2

            lax.fori_loop(112, 128, build, 0)
            return carry

        lax.fori_loop(0, 8, row, 0)

    for m in range(UNITS):
        if m >= NBUF:
            drain_unit()
        h = h0 + 2 * m
        build_unit(vbufs[m % NBUF], h)
        fire_unit(vbufs[m % NBUF], h)
    for m in range(NBUF):
        drain_unit()


def kernel(i, j, relative_attention_bias):
    del i, j
    mesh = plsc.VectorSubcoreMesh(core_axis_name="c", subcore_axis_name="s")
    run = pl.kernel(
        _bias_body,
        out_type=jax.ShapeDtypeStruct((HEADS, SEQ, SEQ), jnp.float32),
        mesh=mesh,
        compiler_params=pltpu.CompilerParams(needs_layout_passes=False),
        scratch_types=[
            pltpu.VMEM((NUM_BUCKETS * HEADS,), jnp.float32),
            pltpu.VMEM((8, VLEN), jnp.float32),
            pltpu.VMEM((8, VLEN), jnp.float32),
            pltpu.VMEM((8, VLEN), jnp.float32),
            pltpu.SemaphoreType.DMA,
        ],
    )
    return run(relative_attention_bias.reshape(-1))
